# baseline (speedup 1.0000x reference)
import functools
import jax
import jax.numpy as jnp
from jax import lax
from jax.experimental import pallas as pl
from jax.experimental.pallas import tpu as pltpu
from jax.experimental.pallas import tpu_sc as plsc

N = 10000
E = 320000
D = 128
NP = 10240
W = 144
NC, NS = 2, 16
NT = NC * NS
C = 128
CH0, CH1 = 160, 0
NCHT = NS * (CH0 + CH1)
EP = NCHT * C
RPT = NP // NS


def _sc_aggregate(xa, src2, dst2, zeros):
    mesh = plsc.VectorSubcoreMesh(core_axis_name="c", subcore_axis_name="s",
                                  num_cores=NC, num_subcores=NS)

    @functools.partial(
        pl.kernel, mesh=mesh,
        out_type=jax.ShapeDtypeStruct((NC, NP, W), jnp.float32),
        scratch_types=[
            pltpu.VMEM((4, C), jnp.int32),
            pltpu.VMEM((4, C), jnp.int32),
            pltpu.VMEM((C, W), jnp.float32),
            pltpu.VMEM((C, W), jnp.float32),
            pltpu.SemaphoreType.DMA,
            pltpu.SemaphoreType.DMA,
            pltpu.SemaphoreType.DMA,
            pltpu.SemaphoreType.DMA,
            pltpu.SemaphoreType.DMA,
            pltpu.SemaphoreType.DMA,
            pltpu.SemaphoreType.DMA,
            pltpu.SemaphoreType.DMA,
            pltpu.VMEM_SHARED((NP, W), jnp.float32),
        ],
        compiler_params=pltpu.CompilerParams(use_tc_tiling_on_sc=False),
    )
    def k(xa_h, src_h, dst_h, z_h, out_h, sidx, didx, r0b, r1b,
          g0, g1, s0, s1, i0, i1, i2, i3, acc):
        rows = [r0b, r1b]
        gs = [g0, g1]
        ss = [s0, s1]
        isem = [i0, i1, i2, i3]
        c = lax.axis_index("c")
        s = lax.axis_index("s")
        r0 = s * RPT
        pltpu.sync_copy(z_h.at[pl.ds(r0, RPT)], acc.at[pl.ds(r0, RPT)])
        plsc.subcore_barrier()

        def run(ebase, nch):
            def si_d(ch, ib):
                pltpu.async_copy(src_h.at[ebase + ch], sidx.at[ib], isem[ib])
                pltpu.async_copy(dst_h.at[ebase + ch], didx.at[ib], isem[ib])

            def wi(ch, ib):
                pltpu.make_async_copy(src_h.at[ebase + ch], sidx.at[ib],
                                      isem[ib]).wait()
                pltpu.make_async_copy(dst_h.at[ebase + ch], didx.at[ib],
                                      isem[ib]).wait()

            def sg(ib, b):
                pltpu.async_copy(xa_h.at[sidx.at[ib]], rows[b], gs[b])

            def wg(ib, b):
                pltpu.make_async_copy(xa_h.at[sidx.at[ib]], rows[b],
                                      gs[b]).wait()

            def sc_(ib, b):
                pltpu.async_copy(rows[b], acc.at[didx.at[ib]], ss[b],
                                 add=True)

            def ws(ib, b):
                pltpu.make_async_copy(rows[b], acc.at[didx.at[ib]],
                                      ss[b]).wait()

            si_d(0, 0); si_d(1, 1); si_d(2, 2)
            wi(0, 0)
            sg(0, 0)
            wi(1, 1)
            wg(0, 0); sc_(0, 0); sg(1, 1); si_d(3, 3)
            wg(1, 1); sc_(1, 1); ws(0, 0); wi(2, 2); sg(2, 0)
            si_d(4, 0)

            def quad(q, carry):
                base = q * 4 + 2
                for b in range(4):
                    ch = base + b
                    ib = (2 + b) % 4
                    rb = b % 2
                    wg(ib, rb)
                    sc_(ib, rb)
                    ws((ib - 1) % 4, 1 - rb)
                    wi(ch + 1, (ib + 1) % 4)
                    sg((ib + 1) % 4, 1 - rb)
                    si_d(ch + 3, (ib + 3) % 4)
                return carry

            lax.fori_loop(0, (nch - 8) // 4, quad, 0)

            for ch in range(nch - 6, nch):
                ib = ch % 4
                rb = ch % 2
                wg(ib, rb)
                sc_(ib, rb)
                ws((ib - 1) % 4, 1 - rb)
                if ch + 1 < nch:
                    wi(ch + 1, (ch + 1) % 4)
                    sg((ch + 1) % 4, 1 - rb)
                if ch + 3 < nch:
                    si_d(ch + 3, (ch + 3) % 4)
            ws((nch - 1) % 4, (nch - 1) % 2)

        @pl.when(c == 0)
        def _():
            run(s * CH0, CH0)

        if CH1:
            @pl.when(c == 1)
            def _():
                run(NS * CH0 + s * CH1, CH1)

        plsc.subcore_barrier()
        pltpu.sync_copy(acc.at[pl.ds(r0, RPT)],
                        out_h.at[c].at[pl.ds(r0, RPT)])

    return k(xa, src2, dst2, zeros)


def _tc_dense(p0, p1, root, WlT, bl, WrT, first_layer):
    B = 512
    Wout = W if first_layer else D

    def body(p0_r, p1_r, x_r, wl_r, bl_r, wr_r, o_r):
        s = p0_r[...] + p1_r[...]
        cnt = s[:, D:D + 1]
        mean = s[:, :D] / jnp.maximum(cnt, 1.0)
        h = (jnp.dot(mean, wl_r[...], preferred_element_type=jnp.float32)
             + jnp.dot(x_r[...][:, :D], wr_r[...],
                       preferred_element_type=jnp.float32)
             + bl_r[...])
        if first_layer:
            h = jnp.maximum(h, 0.0)
            ones = jnp.ones((B, 1), jnp.float32)
            zer = jnp.zeros((B, W - D - 1), jnp.float32)
            o_r[...] = jnp.concatenate([h, ones, zer], axis=1)
        else:
            o_r[...] = h

    return pl.pallas_call(
        body,
        grid=(NP // B,),
        in_specs=[
            pl.BlockSpec((B, W), lambda i: (i, 0)),
            pl.BlockSpec((B, W), lambda i: (i, 0)),
            pl.BlockSpec((B, W), lambda i: (i, 0)),
            pl.BlockSpec((D, D), lambda i: (0, 0)),
            pl.BlockSpec((1, D), lambda i: (0, 0)),
            pl.BlockSpec((D, D), lambda i: (0, 0)),
        ],
        out_specs=pl.BlockSpec((B, Wout), lambda i: (i, 0)),
        out_shape=jax.ShapeDtypeStruct((NP, Wout), jnp.float32),
    )(p0, p1, root, WlT, bl, WrT)


def kernel(x, edge_index, W1l, b1l, W1r, W2l, b2l, W2r):
    f32 = jnp.float32
    xa = jnp.zeros((NP, W), f32).at[:N, :D].set(x).at[:N, D].set(1.0)
    pad = jnp.full((EP - E,), N, jnp.int32)
    src = jnp.concatenate([edge_index[0], pad]).reshape(EP // C, C)
    dst = jnp.concatenate([edge_index[1], pad]).reshape(EP // C, C)
    zeros = jnp.zeros((NP, W), f32)

    p = _sc_aggregate(xa, src, dst, zeros)
    ha = _tc_dense(p[0], p[1], xa, W1l.T, b1l[None, :], W1r.T, True)
    q = _sc_aggregate(ha, src, dst, zeros)
    out = _tc_dense(q[0], q[1], ha, W2l.T, b2l[None, :], W2r.T, False)
    return out[:N]


# speedup vs baseline: 3.0158x; 3.0158x over previous
import functools
import jax
import jax.numpy as jnp
from jax import lax
from jax.experimental import pallas as pl
from jax.experimental.pallas import tpu as pltpu
from jax.experimental.pallas import tpu_sc as plsc

N = 10000
E = 320000
D = 128
NP = 10240
W = 144
NC, NS = 2, 16
NT = NC * NS
C = 128
CH0, CH1 = 80, 80
NCHT = NS * (CH0 + CH1)
EP = NCHT * C
RPT = NP // NS


def _sc_aggregate(xa, src2, dst2, zeros):
    mesh = plsc.VectorSubcoreMesh(core_axis_name="c", subcore_axis_name="s",
                                  num_cores=NC, num_subcores=NS)

    @functools.partial(
        pl.kernel, mesh=mesh,
        out_type=jax.ShapeDtypeStruct((NC, NP, W), jnp.float32),
        scratch_types=[
            pltpu.VMEM((4, C), jnp.int32),
            pltpu.VMEM((4, C), jnp.int32),
            pltpu.VMEM((C, W), jnp.float32),
            pltpu.VMEM((C, W), jnp.float32),
            pltpu.SemaphoreType.DMA,
            pltpu.SemaphoreType.DMA,
            pltpu.SemaphoreType.DMA,
            pltpu.SemaphoreType.DMA,
            pltpu.SemaphoreType.DMA,
            pltpu.SemaphoreType.DMA,
            pltpu.SemaphoreType.DMA,
            pltpu.SemaphoreType.DMA,
            pltpu.VMEM_SHARED((NP, W), jnp.float32),
        ],
        compiler_params=pltpu.CompilerParams(use_tc_tiling_on_sc=False),
    )
    def k(xa_h, src_h, dst_h, z_h, out_h, sidx, didx, r0b, r1b,
          g0, g1, s0, s1, i0, i1, i2, i3, acc):
        rows = [r0b, r1b]
        gs = [g0, g1]
        ss = [s0, s1]
        isem = [i0, i1, i2, i3]
        c = lax.axis_index("c")
        s = lax.axis_index("s")
        r0 = s * RPT
        pltpu.sync_copy(z_h.at[pl.ds(r0, RPT)], acc.at[pl.ds(r0, RPT)])
        plsc.subcore_barrier()

        def run(ebase, nch):
            def si_d(ch, ib):
                pltpu.async_copy(src_h.at[ebase + ch], sidx.at[ib], isem[ib])
                pltpu.async_copy(dst_h.at[ebase + ch], didx.at[ib], isem[ib])

            def wi(ch, ib):
                pltpu.make_async_copy(src_h.at[ebase + ch], sidx.at[ib],
                                      isem[ib]).wait()
                pltpu.make_async_copy(dst_h.at[ebase + ch], didx.at[ib],
                                      isem[ib]).wait()

            def sg(ib, b):
                pltpu.async_copy(xa_h.at[sidx.at[ib]], rows[b], gs[b])

            def wg(ib, b):
                pltpu.make_async_copy(xa_h.at[sidx.at[ib]], rows[b],
                                      gs[b]).wait()

            def sc_(ib, b):
                pltpu.async_copy(rows[b], acc.at[didx.at[ib]], ss[b],
                                 add=True)

            def ws(ib, b):
                pltpu.make_async_copy(rows[b], acc.at[didx.at[ib]],
                                      ss[b]).wait()

            si_d(0, 0); si_d(1, 1); si_d(2, 2)
            wi(0, 0)
            sg(0, 0)
            wi(1, 1)
            wg(0, 0); sc_(0, 0); sg(1, 1); si_d(3, 3)
            wg(1, 1); sc_(1, 1); ws(0, 0); wi(2, 2); sg(2, 0)
            si_d(4, 0)

            def quad(q, carry):
                base = q * 4 + 2
                for b in range(4):
                    ch = base + b
                    ib = (2 + b) % 4
                    rb = b % 2
                    wg(ib, rb)
                    sc_(ib, rb)
                    ws((ib - 1) % 4, 1 - rb)
                    wi(ch + 1, (ib + 1) % 4)
                    sg((ib + 1) % 4, 1 - rb)
                    si_d(ch + 3, (ib + 3) % 4)
                return carry

            lax.fori_loop(0, (nch - 8) // 4, quad, 0)

            for ch in range(nch - 6, nch):
                ib = ch % 4
                rb = ch % 2
                wg(ib, rb)
                sc_(ib, rb)
                ws((ib - 1) % 4, 1 - rb)
                if ch + 1 < nch:
                    wi(ch + 1, (ch + 1) % 4)
                    sg((ch + 1) % 4, 1 - rb)
                if ch + 3 < nch:
                    si_d(ch + 3, (ch + 3) % 4)
            ws((nch - 1) % 4, (nch - 1) % 2)

        @pl.when(c == 0)
        def _():
            run(s * CH0, CH0)

        if CH1:
            @pl.when(c == 1)
            def _():
                run(NS * CH0 + s * CH1, CH1)

        plsc.subcore_barrier()
        pltpu.sync_copy(acc.at[pl.ds(r0, RPT)],
                        out_h.at[c].at[pl.ds(r0, RPT)])

    return k(xa, src2, dst2, zeros)


def _tc_dense(p0, p1, root, WlT, bl, WrT, first_layer):
    B = 512
    Wout = W if first_layer else D

    def body(p0_r, p1_r, x_r, wl_r, bl_r, wr_r, o_r):
        s = p0_r[...] + p1_r[...]
        cnt = s[:, D:D + 1]
        mean = s[:, :D] / jnp.maximum(cnt, 1.0)
        h = (jnp.dot(mean, wl_r[...], preferred_element_type=jnp.float32)
             + jnp.dot(x_r[...][:, :D], wr_r[...],
                       preferred_element_type=jnp.float32)
             + bl_r[...])
        if first_layer:
            h = jnp.maximum(h, 0.0)
            ones = jnp.ones((B, 1), jnp.float32)
            zer = jnp.zeros((B, W - D - 1), jnp.float32)
            o_r[...] = jnp.concatenate([h, ones, zer], axis=1)
        else:
            o_r[...] = h

    return pl.pallas_call(
        body,
        grid=(NP // B,),
        in_specs=[
            pl.BlockSpec((B, W), lambda i: (i, 0)),
            pl.BlockSpec((B, W), lambda i: (i, 0)),
            pl.BlockSpec((B, W), lambda i: (i, 0)),
            pl.BlockSpec((D, D), lambda i: (0, 0)),
            pl.BlockSpec((1, D), lambda i: (0, 0)),
            pl.BlockSpec((D, D), lambda i: (0, 0)),
        ],
        out_specs=pl.BlockSpec((B, Wout), lambda i: (i, 0)),
        out_shape=jax.ShapeDtypeStruct((NP, Wout), jnp.float32),
    )(p0, p1, root, WlT, bl, WrT)


def kernel(x, edge_index, W1l, b1l, W1r, W2l, b2l, W2r):
    f32 = jnp.float32
    xa = jnp.zeros((NP, W), f32).at[:N, :D].set(x).at[:N, D].set(1.0)
    pad = N + (jnp.arange(EP - E, dtype=jnp.int32) % (NP - N))
    src = jnp.concatenate([edge_index[0], pad]).reshape(EP // C, C)
    dst = jnp.concatenate([edge_index[1], pad]).reshape(EP // C, C)
    zeros = jnp.zeros((NP, W), f32)

    p = _sc_aggregate(xa, src, dst, zeros)
    ha = _tc_dense(p[0], p[1], xa, W1l.T, b1l[None, :], W1r.T, True)
    q = _sc_aggregate(ha, src, dst, zeros)
    out = _tc_dense(q[0], q[1], ha, W2l.T, b2l[None, :], W2r.T, False)
    return out[:N]


# speedup vs baseline: 3.2070x; 1.0634x over previous
import functools
import jax
import jax.numpy as jnp
from jax import lax
from jax.experimental import pallas as pl
from jax.experimental.pallas import tpu as pltpu
from jax.experimental.pallas import tpu_sc as plsc

N = 10000
E = 320000
D = 128
NP = 10240
W = 144
NC, NS = 2, 16
NT = NC * NS
C = 128
CH0, CH1 = 80, 80
NCHT = NS * (CH0 + CH1)
EP = NCHT * C
RPT = NP // NS


def _sc_aggregate(xa, src2, dst2, zeros):
    mesh = plsc.VectorSubcoreMesh(core_axis_name="c", subcore_axis_name="s",
                                  num_cores=NC, num_subcores=NS)

    @functools.partial(
        pl.kernel, mesh=mesh,
        out_type=jax.ShapeDtypeStruct((NC, NP, W), jnp.float32),
        scratch_types=[
            pltpu.VMEM((4, C), jnp.int32),
            pltpu.VMEM((4, C), jnp.int32),
            pltpu.VMEM((C, W), jnp.float32),
            pltpu.VMEM((C, W), jnp.float32),
            pltpu.SemaphoreType.DMA,
            pltpu.SemaphoreType.DMA,
            pltpu.SemaphoreType.DMA,
            pltpu.SemaphoreType.DMA,
            pltpu.SemaphoreType.DMA,
            pltpu.SemaphoreType.DMA,
            pltpu.SemaphoreType.DMA,
            pltpu.SemaphoreType.DMA,
            pltpu.VMEM_SHARED((NP, W), jnp.float32),
        ],
        compiler_params=pltpu.CompilerParams(use_tc_tiling_on_sc=False),
    )
    def k(xa_h, src_h, dst_h, z_h, out_h, sidx, didx, r0b, r1b,
          g0, g1, s0, s1, i0, i1, i2, i3, acc):
        rows = [r0b, r1b]
        gs = [g0, g1]
        ss = [s0, s1]
        isem = [i0, i1, i2, i3]
        c = lax.axis_index("c")
        s = lax.axis_index("s")
        r0 = s * RPT
        pltpu.sync_copy(z_h.at[pl.ds(r0, RPT)], acc.at[pl.ds(r0, RPT)])
        plsc.subcore_barrier()

        def run(ebase, nch):
            def si_d(ch, ib):
                pltpu.async_copy(src_h.at[ebase + ch], sidx.at[ib], isem[ib])
                pltpu.async_copy(dst_h.at[ebase + ch], didx.at[ib], isem[ib])

            def wi(ch, ib):
                pltpu.make_async_copy(src_h.at[ebase + ch], sidx.at[ib],
                                      isem[ib]).wait()
                pltpu.make_async_copy(dst_h.at[ebase + ch], didx.at[ib],
                                      isem[ib]).wait()

            def sg(ib, b):
                pltpu.async_copy(xa_h.at[sidx.at[ib]], rows[b], gs[b])

            def wg(ib, b):
                pltpu.make_async_copy(xa_h.at[sidx.at[ib]], rows[b],
                                      gs[b]).wait()

            def sc_(ib, b):
                pltpu.async_copy(rows[b], acc.at[didx.at[ib]], ss[b],
                                 add=True)

            def ws(ib, b):
                pltpu.make_async_copy(rows[b], acc.at[didx.at[ib]],
                                      ss[b]).wait()

            si_d(0, 0); si_d(1, 1); si_d(2, 2)
            wi(0, 0)
            sg(0, 0)
            wi(1, 1)
            wg(0, 0); sc_(0, 0); sg(1, 1); si_d(3, 3)
            wg(1, 1); sc_(1, 1); ws(0, 0); wi(2, 2); sg(2, 0)
            si_d(4, 0)

            def quad(q, carry):
                base = q * 4 + 2
                for b in range(4):
                    ch = base + b
                    ib = (2 + b) % 4
                    rb = b % 2
                    wg(ib, rb)
                    sc_(ib, rb)
                    ws((ib - 1) % 4, 1 - rb)
                    wi(ch + 1, (ib + 1) % 4)
                    sg((ib + 1) % 4, 1 - rb)
                    si_d(ch + 3, (ib + 3) % 4)
                return carry

            lax.fori_loop(0, (nch - 8) // 4, quad, 0)

            for ch in range(nch - 6, nch):
                ib = ch % 4
                rb = ch % 2
                wg(ib, rb)
                sc_(ib, rb)
                ws((ib - 1) % 4, 1 - rb)
                if ch + 1 < nch:
                    wi(ch + 1, (ch + 1) % 4)
                    sg((ch + 1) % 4, 1 - rb)
                if ch + 3 < nch:
                    si_d(ch + 3, (ch + 3) % 4)
            ws((nch - 1) % 4, (nch - 1) % 2)

        @pl.when(c == 0)
        def _():
            run(s * CH0, CH0)

        if CH1:
            @pl.when(c == 1)
            def _():
                run(NS * CH0 + s * CH1, CH1)

        plsc.subcore_barrier()
        pltpu.sync_copy(acc.at[pl.ds(r0, RPT)],
                        out_h.at[c].at[pl.ds(r0, RPT)])

    return k(xa, src2, dst2, zeros)


def _tc_dense(p, root, WlT, bl, WrT, first_layer):
    B = 512 if first_layer else 400
    NR = NP if first_layer else N
    Wout = W if first_layer else D

    def body(p0_r, p1_r, x_r, wl_r, bl_r, wr_r, o_r):
        s = p0_r[0] + p1_r[0]
        cnt = s[:, D:D + 1]
        mean = s[:, :D] / jnp.maximum(cnt, 1.0)
        h = (jnp.dot(mean, wl_r[...], preferred_element_type=jnp.float32)
             + jnp.dot(x_r[...][:, :D], wr_r[...],
                       preferred_element_type=jnp.float32)
             + bl_r[...])
        if first_layer:
            h = jnp.maximum(h, 0.0)
            ones = jnp.ones((B, 1), jnp.float32)
            zer = jnp.zeros((B, W - D - 1), jnp.float32)
            o_r[...] = jnp.concatenate([h, ones, zer], axis=1)
        else:
            o_r[...] = h

    return pl.pallas_call(
        body,
        grid=(NR // B,),
        in_specs=[
            pl.BlockSpec((1, B, W), lambda i: (0, i, 0)),
            pl.BlockSpec((1, B, W), lambda i: (1, i, 0)),
            pl.BlockSpec((B, W), lambda i: (i, 0)),
            pl.BlockSpec((D, D), lambda i: (0, 0)),
            pl.BlockSpec((1, D), lambda i: (0, 0)),
            pl.BlockSpec((D, D), lambda i: (0, 0)),
        ],
        out_specs=pl.BlockSpec((B, Wout), lambda i: (i, 0)),
        out_shape=jax.ShapeDtypeStruct((NR, Wout), jnp.float32),
    )(p, p, root, WlT, bl, WrT)


def kernel(x, edge_index, W1l, b1l, W1r, W2l, b2l, W2r):
    f32 = jnp.float32
    xa = jnp.zeros((NP, W), f32).at[:N, :D].set(x).at[:N, D].set(1.0)
    pad = N + (jnp.arange(EP - E, dtype=jnp.int32) % (NP - N))
    src = jnp.concatenate([edge_index[0], pad]).reshape(EP // C, C)
    dst = jnp.concatenate([edge_index[1], pad]).reshape(EP // C, C)
    zeros = jnp.zeros((NP, W), f32)

    p = _sc_aggregate(xa, src, dst, zeros)
    ha = _tc_dense(p, xa, W1l.T, b1l[None, :], W1r.T, True)
    q = _sc_aggregate(ha, src, dst, zeros)
    return _tc_dense(q, ha, W2l.T, b2l[None, :], W2r.T, False)
